# baseline (speedup 1.0000x reference)
import functools

import jax
import jax.numpy as jnp
from jax import lax
from jax.experimental import pallas as pl
from jax.experimental.pallas import tpu as pltpu
from jax.experimental.pallas import tpu_sc as plsc

_NB = 4
_NO = 2


def _make_gather(V, D, B, NC, NS):
    NW = NC * NS
    C = 128
    L = 16
    G = C // L
    b_per_w = B // NW
    K = b_per_w // C
    assert b_per_w * NW == B and K * C == b_per_w and K % _NB == 0

    mesh = plsc.VectorSubcoreMesh(core_axis_name="c", subcore_axis_name="s")

    @functools.partial(
        pl.kernel,
        mesh=mesh,
        compiler_params=pltpu.CompilerParams(needs_layout_passes=False),
        out_type=jax.ShapeDtypeStruct((D, B), jnp.float32),
        scratch_types=[
            pltpu.VMEM((K, C), jnp.int32),
            [pltpu.VMEM((C,), jnp.int32) for _ in range(_NB)],
            [pltpu.VMEM((C, 2 * D), jnp.float32) for _ in range(_NB)],
            [pltpu.VMEM((D, C), jnp.float32) for _ in range(_NO)],
            [pltpu.SemaphoreType.DMA for _ in range(_NB)],
            [pltpu.SemaphoreType.DMA for _ in range(_NO)],
        ],
    )
    def gather_kernel(r2d_hbm, idx_hbm, outT_hbm,
                      idx_v, pbufs, bufs, obufs, gsems, wsems):
        wid = lax.axis_index("s") * NC + lax.axis_index("c")
        base = wid * b_per_w
        pltpu.sync_copy(idx_hbm.at[wid], idx_v)

        def fire_gather(g, b):
            for gi in range(G):
                pbufs[b][pl.ds(gi * L, L)] = (
                    lax.shift_right_logical(idx_v[g, pl.ds(gi * L, L)], 1))
            pltpu.make_async_copy(
                r2d_hbm.at[pbufs[b]], bufs[b], gsems[b]).start()

        def gather_wait(g, b):
            pltpu.make_async_copy(
                r2d_hbm.at[pbufs[b]], bufs[b], gsems[b]).wait()

        def writeback(g, o):
            return pltpu.make_async_copy(
                obufs[o], outT_hbm.at[:, pl.ds(base + g * C, C)], wsems[o])

        rows = [lax.iota(jnp.int32, L) + gi * L for gi in range(G)]

        for b in range(_NB - 1):
            fire_gather(b, b)

        @pl.loop(0, K, step=_NB)
        def _lap(j):
            for b in range(_NB):
                g = j + b
                o = b % _NO
                bp = (b - 1) % _NB

                if b == 0:
                    @pl.when(j >= 1)
                    def _wbwait0():
                        writeback(g - 1, (_NO - 1)).wait()
                else:
                    writeback(g - 1, (b - 1) % _NO).wait()
                @pl.when(g + _NB - 1 < K)
                def _refill():
                    fire_gather(g + _NB - 1, bp)

                gather_wait(g, b)
                cols = [(idx_v[g, pl.ds(gi * L, L)] & 1) * D for gi in range(G)]

                @pl.loop(0, D)
                def _feat(c):
                    for gi in range(G):
                        v = plsc.load_gather(bufs[b], [rows[gi], cols[gi] + c])
                        obufs[o][c, pl.ds(gi * L, L)] = v

                writeback(g, o).start()

        writeback(K - 1, (K - 1) % _NO).wait()

    return gather_kernel


def kernel(input, dim, index):
    table = input
    V, D = table.shape
    (B,) = index.shape
    info = plsc.get_sparse_core_info()
    NC, NS = info.num_cores, info.num_subcores
    NW = NC * NS
    C = 128
    idx3 = index.astype(jnp.int32).reshape(NW, (B // NW) // C, C)
    r2d = table.reshape(V // 2, 2 * D)
    outT = _make_gather(V, D, B, NC, NS)(r2d, idx3)
    return outT.T


# speedup vs baseline: 1.1444x; 1.1444x over previous
import functools

import jax
import jax.numpy as jnp
from jax import lax
from jax.experimental import pallas as pl
from jax.experimental.pallas import tpu as pltpu
from jax.experimental.pallas import tpu_sc as plsc

_NB = 4
_NO = 2


def _make_gather(V, D, B, NC, NS):
    NW = NC * NS
    C = 128
    L = 16
    G = C // L
    b_per_w = B // NW
    K = b_per_w // C
    assert b_per_w * NW == B and K * C == b_per_w and K % _NB == 0

    mesh = plsc.VectorSubcoreMesh(core_axis_name="c", subcore_axis_name="s")

    @functools.partial(
        pl.kernel,
        mesh=mesh,
        compiler_params=pltpu.CompilerParams(needs_layout_passes=False),
        out_type=jax.ShapeDtypeStruct((D, B), jnp.float32),
        scratch_types=[
            pltpu.VMEM((K, C), jnp.int32),
            [pltpu.VMEM((C,), jnp.int32) for _ in range(_NB)],
            [pltpu.VMEM((C, 2 * D), jnp.float32) for _ in range(_NB)],
            [pltpu.VMEM((D, C), jnp.float32) for _ in range(_NO)],
            [pltpu.SemaphoreType.DMA for _ in range(_NB)],
            [pltpu.SemaphoreType.DMA for _ in range(_NO)],
        ],
    )
    def gather_kernel(r2d_hbm, idx_hbm, outT_hbm,
                      idx_v, pbufs, bufs, obufs, gsems, wsems):
        wid = lax.axis_index("s") * NC + lax.axis_index("c")
        base = wid * b_per_w
        pltpu.sync_copy(idx_hbm.at[wid], idx_v)

        def fire_gather(g, b):
            for gi in range(G):
                pbufs[b][pl.ds(gi * L, L)] = (
                    lax.shift_right_logical(idx_v[g, pl.ds(gi * L, L)], 1))
            pltpu.make_async_copy(
                r2d_hbm.at[pbufs[b]], bufs[b], gsems[b]).start()

        def gather_wait(g, b):
            pltpu.make_async_copy(
                r2d_hbm.at[pbufs[b]], bufs[b], gsems[b]).wait()

        def writeback(g, o):
            return pltpu.make_async_copy(
                obufs[o], outT_hbm.at[:, pl.ds(base + g * C, C)], wsems[o])

        rows = [lax.iota(jnp.int32, L) + gi * L for gi in range(G)]

        for b in range(_NB - 1):
            fire_gather(b, b)

        @pl.loop(0, K, step=_NB)
        def _lap(j):
            for b in range(_NB):
                g = j + b
                o = b % _NO
                bp = (b - 1) % _NB

                if b == 0:
                    @pl.when(j >= 1)
                    def _wbwait0():
                        writeback(g - 1, (_NO - 1)).wait()
                else:
                    writeback(g - 1, (b - 1) % _NO).wait()
                @pl.when(g + _NB - 1 < K)
                def _refill():
                    fire_gather(g + _NB - 1, bp)

                gather_wait(g, b)
                cols = [(idx_v[g, pl.ds(gi * L, L)] & 1) * D for gi in range(G)]

                @pl.loop(0, D, unroll=4)
                def _feat(c):
                    vs = [plsc.load_gather(bufs[b], [rows[gi], cols[gi] + c])
                          for gi in range(G)]
                    for gi in range(G):
                        obufs[o][c, pl.ds(gi * L, L)] = vs[gi]

                writeback(g, o).start()

        writeback(K - 1, (K - 1) % _NO).wait()

    return gather_kernel


def kernel(input, dim, index):
    table = input
    V, D = table.shape
    (B,) = index.shape
    info = plsc.get_sparse_core_info()
    NC, NS = info.num_cores, info.num_subcores
    NW = NC * NS
    C = 128
    idx3 = index.astype(jnp.int32).reshape(NW, (B // NW) // C, C)
    r2d = table.reshape(V // 2, 2 * D)
    outT = _make_gather(V, D, B, NC, NS)(r2d, idx3)
    return outT.T


# speedup vs baseline: 1.7060x; 1.4907x over previous
import functools

import jax
import jax.numpy as jnp
from jax import lax
from jax.experimental import pallas as pl
from jax.experimental.pallas import tpu as pltpu
from jax.experimental.pallas import tpu_sc as plsc

_NB = 4
_NO = 2


def _make_gather(V, D, B, NC, NS):
    NW = NC * NS
    C = 128
    L = 16
    G = C // L
    b_per_w = B // NW
    K = b_per_w // C
    assert b_per_w * NW == B and K * C == b_per_w and K % _NB == 0

    mesh = plsc.VectorSubcoreMesh(core_axis_name="c", subcore_axis_name="s")

    @functools.partial(
        pl.kernel,
        mesh=mesh,
        compiler_params=pltpu.CompilerParams(needs_layout_passes=False),
        out_type=jax.ShapeDtypeStruct((D, B), jnp.float32),
        scratch_types=[
            pltpu.VMEM((K, C), jnp.int32),
            [pltpu.VMEM((C,), jnp.int32) for _ in range(_NB)],
            [pltpu.VMEM((C, 2 * D), jnp.float32) for _ in range(_NB)],
            [pltpu.VMEM((D, C), jnp.float32) for _ in range(_NO)],
            [pltpu.SemaphoreType.DMA for _ in range(_NB)],
            [pltpu.SemaphoreType.DMA for _ in range(_NO)],
        ],
    )
    def gather_kernel(r2d_hbm, idx_hbm, outT_hbm,
                      idx_v, pbufs, bufs, obufs, gsems, wsems):
        wid = lax.axis_index("s") * NC + lax.axis_index("c")
        base = wid * b_per_w
        pltpu.sync_copy(idx_hbm.at[wid], idx_v)

        def fire_gather(g, b):
            for gi in range(G):
                pbufs[b][pl.ds(gi * L, L)] = (
                    lax.shift_right_logical(idx_v[g, pl.ds(gi * L, L)], 1))
            pltpu.make_async_copy(
                r2d_hbm.at[pbufs[b]], bufs[b], gsems[b]).start()

        def gather_wait(g, b):
            pltpu.make_async_copy(
                r2d_hbm.at[pbufs[b]], bufs[b], gsems[b]).wait()

        def writeback(g, o):
            return pltpu.make_async_copy(
                obufs[o], outT_hbm.at[:, pl.ds(base + g * C, C)], wsems[o])

        rows = [lax.iota(jnp.int32, L) + gi * L for gi in range(G)]

        for b in range(_NB - 1):
            fire_gather(b, b)

        @pl.loop(0, K, step=_NB)
        def _lap(j):
            for b in range(_NB):
                g = j + b
                o = b % _NO
                bp = (b - 1) % _NB

                if b == 0:
                    @pl.when(j >= 1)
                    def _wbwait0():
                        writeback(g - 1, (_NO - 1)).wait()
                else:
                    writeback(g - 1, (b - 1) % _NO).wait()
                @pl.when(g + _NB - 1 < K)
                def _refill():
                    fire_gather(g + _NB - 1, bp)

                gather_wait(g, b)
                cols = [(idx_v[g, pl.ds(gi * L, L)] & 1) * D for gi in range(G)]

                @pl.loop(0, L, unroll=2)
                def _diag(d):
                    cvec = (lax.iota(jnp.int32, L) + d) & (L - 1)
                    for gi in range(G):
                        for k in range(D // L):
                            fvec = cvec + (k * L)
                            v = plsc.load_gather(
                                bufs[b], [rows[gi], cols[gi] + fvec])
                            plsc.store_scatter(
                                obufs[o], [fvec, rows[gi]], v)

                writeback(g, o).start()

        writeback(K - 1, (K - 1) % _NO).wait()

    return gather_kernel


def kernel(input, dim, index):
    table = input
    V, D = table.shape
    (B,) = index.shape
    info = plsc.get_sparse_core_info()
    NC, NS = info.num_cores, info.num_subcores
    NW = NC * NS
    C = 128
    idx3 = index.astype(jnp.int32).reshape(NW, (B // NW) // C, C)
    r2d = table.reshape(V // 2, 2 * D)
    outT = _make_gather(V, D, B, NC, NS)(r2d, idx3)
    return outT.T


# speedup vs baseline: 2.2990x; 1.3475x over previous
import functools

import jax
import jax.numpy as jnp
from jax import lax
from jax.experimental import pallas as pl
from jax.experimental.pallas import tpu as pltpu
from jax.experimental.pallas import tpu_sc as plsc

_NB = 4
_NO = 2


def _make_gather(V, D, B, NC, NS):
    NW = NC * NS
    C = 128
    L = 16
    G = C // L
    b_per_w = B // NW
    K = b_per_w // C
    assert b_per_w * NW == B and K * C == b_per_w and K % _NB == 0

    mesh = plsc.VectorSubcoreMesh(core_axis_name="c", subcore_axis_name="s")

    @functools.partial(
        pl.kernel,
        mesh=mesh,
        compiler_params=pltpu.CompilerParams(needs_layout_passes=False),
        out_type=jax.ShapeDtypeStruct((D, B), jnp.float32),
        scratch_types=[
            pltpu.VMEM((K, C), jnp.int32),
            [pltpu.VMEM((C,), jnp.int32) for _ in range(_NB)],
            [pltpu.VMEM((C, 2 * D), jnp.float32) for _ in range(_NB)],
            [pltpu.VMEM((D, C), jnp.float32) for _ in range(_NO)],
            [pltpu.SemaphoreType.DMA for _ in range(_NB)],
            [pltpu.SemaphoreType.DMA for _ in range(_NO)],
        ],
    )
    def gather_kernel(r2d_hbm, idx_hbm, outT_hbm,
                      idx_v, pbufs, bufs, obufs, gsems, wsems):
        wid = lax.axis_index("s") * NC + lax.axis_index("c")
        base = wid * b_per_w
        pltpu.sync_copy(idx_hbm.at[wid], idx_v)

        def fire_gather(g, b):
            for gi in range(G):
                pbufs[b][pl.ds(gi * L, L)] = (
                    lax.shift_right_logical(idx_v[g, pl.ds(gi * L, L)], 1))
            pltpu.make_async_copy(
                r2d_hbm.at[pbufs[b]], bufs[b], gsems[b]).start()

        def gather_wait(g, b):
            pltpu.make_async_copy(
                r2d_hbm.at[pbufs[b]], bufs[b], gsems[b]).wait()

        def writeback(g, o):
            return pltpu.make_async_copy(
                obufs[o], outT_hbm.at[:, pl.ds(base + g * C, C)], wsems[o])

        rows = [lax.iota(jnp.int32, L) + gi * L for gi in range(G)]

        for b in range(_NB - 1):
            fire_gather(b, b)

        @pl.loop(0, K, step=_NB)
        def _lap(j):
            for b in range(_NB):
                g = j + b
                o = b % _NO
                bp = (b - 1) % _NB

                if b == 0:
                    @pl.when(j >= 1)
                    def _wbwait0():
                        writeback(g - 1, (_NO - 1)).wait()
                else:
                    writeback(g - 1, (b - 1) % _NO).wait()
                @pl.when(g + _NB - 1 < K)
                def _refill():
                    fire_gather(g + _NB - 1, bp)

                gather_wait(g, b)
                cols = [(idx_v[g, pl.ds(gi * L, L)] & 1) * D for gi in range(G)]

                @pl.loop(0, L)
                def _diag(d):
                    cvec = (lax.iota(jnp.int32, L) + d) & (L - 1)
                    fvecs = [cvec + (k * L) for k in range(D // L)]
                    vs = [plsc.load_gather(
                              bufs[b], [rows[gi], cols[gi] + fvecs[k]])
                          for gi in range(G) for k in range(D // L)]
                    i = 0
                    for gi in range(G):
                        for k in range(D // L):
                            plsc.store_scatter(
                                obufs[o], [fvecs[k], rows[gi]], vs[i])
                            i += 1

                writeback(g, o).start()

        writeback(K - 1, (K - 1) % _NO).wait()

    return gather_kernel


def kernel(input, dim, index):
    table = input
    V, D = table.shape
    (B,) = index.shape
    info = plsc.get_sparse_core_info()
    NC, NS = info.num_cores, info.num_subcores
    NW = NC * NS
    C = 128
    idx3 = index.astype(jnp.int32).reshape(NW, (B // NW) // C, C)
    r2d = table.reshape(V // 2, 2 * D)
    outT = _make_gather(V, D, B, NC, NS)(r2d, idx3)
    return outT.T
